# baseline (speedup 1.0000x reference)
import jax
import jax.numpy as jnp
from jax.experimental import pallas as pl
from jax.experimental.pallas import tpu as pltpu

N = 2048
DIM = 768
HEAD_DIM = 64
E_ATTN = 16
E_FFD = 8
FFD_K = 2
N_HEADS = 12
SCALE = HEAD_DIM ** -0.5
TILE = 512
NT = N // TILE


def _topk_gates_dense(logits, k):
    t, e = logits.shape
    eidx = jax.lax.broadcasted_iota(jnp.int32, (t, e), 1)
    rank = jnp.zeros((t, e), jnp.int32)
    for j in range(e):
        lj = logits[:, j:j + 1]
        beats = (lj > logits) | ((lj == logits) & (j < eidx))
        rank += beats.astype(jnp.int32)
    mask = rank < k
    m = jnp.max(logits, axis=-1, keepdims=True)
    ex = jnp.where(mask, jnp.exp(logits - m), 0.0)
    return ex / jnp.sum(ex, axis=-1, keepdims=True)


def _layer_norm(x, g, b):
    mu = jnp.mean(x, axis=-1, keepdims=True)
    var = jnp.mean((x - mu) ** 2, axis=-1, keepdims=True)
    return (x - mu) * jax.lax.rsqrt(var + 1e-5) * g + b


def _kernel(x_ref, g1_ref, b1l_ref, wga_ref, wkv_ref, bkv_ref, wmap_ref,
            bmap_ref, wout_ref, bout_ref, g2_ref, b2l_ref, wgm_ref,
            w1_ref, b1_ref, w2_ref, b2f_ref,
            out_ref,
            x1_s, qall_s, k_s, v_s, g16_s, xn2_s, g8_s, o16_s):
    t = pl.program_id(0)

    @pl.when(t < NT)
    def _pre():
        rows = pl.ds(t * TILE, TILE)
        x = x_ref[...]
        x1_s[rows, :] = x
        xn = _layer_norm(x, g1_ref[...], b1l_ref[...])
        xnb = xn.astype(jnp.bfloat16)
        logits = jnp.dot(xn, wga_ref[...], preferred_element_type=jnp.float32)
        g16_s[rows, :] = _topk_gates_dense(logits, N_HEADS)
        kv = jnp.dot(xnb, wkv_ref[...].astype(jnp.bfloat16),
                     preferred_element_type=jnp.float32) + bkv_ref[...]
        k_s[rows, :] = kv[:, :HEAD_DIM].astype(jnp.bfloat16)
        v_s[rows, :] = kv[:, HEAD_DIM:].astype(jnp.bfloat16)
        qall = jnp.dot(xnb, wmap_ref[...].astype(jnp.bfloat16),
                       preferred_element_type=jnp.float32) + bmap_ref[...]
        qall_s[rows, :] = (qall * (SCALE * 1.4426950408889634)
                           ).astype(jnp.bfloat16)

    @pl.when((t >= NT) & (t < 2 * NT))
    def _attn():
        rows = pl.ds((t - NT) * TILE, TILE)
        qall = qall_s[rows, :]
        k = k_s[...]
        v = v_s[...]
        g16 = g16_s[rows, :]
        for e in range(E_ATTN):
            q = qall[:, e * HEAD_DIM:(e + 1) * HEAD_DIM]
            s = jax.lax.dot_general(q, k, (((1,), (1,)), ((), ())),
                                    preferred_element_type=jnp.float32)
            p = jnp.exp2(s)
            denom = jnp.sum(p, axis=-1, keepdims=True)
            scale = g16[:, e:e + 1] / denom
            o = jnp.dot(p.astype(jnp.bfloat16), v,
                        preferred_element_type=jnp.float32)
            o16_s[:, e * HEAD_DIM:(e + 1) * HEAD_DIM] = (
                o * scale).astype(jnp.bfloat16)
        y = jnp.dot(o16_s[...], wout_ref[...].astype(jnp.bfloat16),
                    preferred_element_type=jnp.float32)
        y = y + jnp.dot(g16, bout_ref[...],
                        preferred_element_type=jnp.float32)
        x1 = x1_s[rows, :] + y
        x1_s[rows, :] = x1
        xn2 = _layer_norm(x1, g2_ref[...], b2l_ref[...])
        xn2_s[rows, :] = xn2.astype(jnp.bfloat16)
        logits = jnp.dot(xn2, wgm_ref[...], preferred_element_type=jnp.float32)
        g8_s[rows, :] = _topk_gates_dense(logits, FFD_K)

    @pl.when(t >= 2 * NT)
    def _ffn():
        e = t - 2 * NT
        xn2 = xn2_s[...]
        g8 = g8_s[...]
        h = jnp.dot(xn2, w1_ref[0].astype(jnp.bfloat16),
                    preferred_element_type=jnp.float32)
        h = h + b1_ref[0]
        inner = h * (0.7978845608028654 + 0.035677408136300125 * (h * h))
        h = 0.5 * h * (1.0 + jnp.tanh(inner))
        sel = (jax.lax.broadcasted_iota(jnp.int32, (E_FFD, 1), 0) == e
               ).astype(jnp.float32)
        g = jnp.dot(g8, sel, preferred_element_type=jnp.float32)
        hw = (h * g).astype(jnp.bfloat16)
        acc = jnp.dot(hw, w2_ref[0].astype(jnp.bfloat16),
                      preferred_element_type=jnp.float32)

        @pl.when(e == 0)
        def _init():
            out_ref[...] = x1_s[...] + jnp.dot(
                g8, b2f_ref[...], preferred_element_type=jnp.float32) + acc

        @pl.when(e != 0)
        def _acc():
            out_ref[...] = out_ref[...] + acc


def _full(shape):
    n = len(shape)
    return pl.BlockSpec(shape, lambda *_: (0,) * n)


def kernel(x, task_bh, ln1_g, ln1_b, ln2_g, ln2_b, wg_attn, w_map, b_map,
           w_out, b_out, w_kv, b_kv, wg_mlp, w1, b1, w2, b2):
    x2d = x.reshape(N, DIM)
    wg_a = jax.lax.dynamic_index_in_dim(wg_attn, task_bh, 0, keepdims=False)
    wg_m = jax.lax.dynamic_index_in_dim(wg_mlp, task_bh, 0, keepdims=False)
    w_mapf = jnp.transpose(w_map, (1, 0, 2)).reshape(DIM, E_ATTN * HEAD_DIM)
    b_mapf = b_map.reshape(1, E_ATTN * HEAD_DIM)
    w_outf = w_out.reshape(E_ATTN * HEAD_DIM, DIM)

    def _xmap(t):
        return (jnp.minimum(t, NT - 1), 0)

    def _emap3(t):
        return (jnp.clip(t - 2 * NT, 0, E_FFD - 1), 0, 0)

    out = pl.pallas_call(
        _kernel,
        grid=(2 * NT + E_FFD,),
        in_specs=[
            pl.BlockSpec((TILE, DIM), _xmap),
            _full((1, DIM)), _full((1, DIM)),
            _full((DIM, E_ATTN)),
            _full((DIM, 2 * HEAD_DIM)), _full((1, 2 * HEAD_DIM)),
            _full((DIM, E_ATTN * HEAD_DIM)), _full((1, E_ATTN * HEAD_DIM)),
            _full((E_ATTN * HEAD_DIM, DIM)), _full((E_ATTN, DIM)),
            _full((1, DIM)), _full((1, DIM)),
            _full((DIM, E_FFD)),
            pl.BlockSpec((1, DIM, DIM), _emap3),
            pl.BlockSpec((1, 1, DIM), _emap3),
            pl.BlockSpec((1, DIM, DIM), _emap3),
            _full((E_FFD, DIM)),
        ],
        out_specs=_full((N, DIM)),
        out_shape=jax.ShapeDtypeStruct((N, DIM), jnp.float32),
        scratch_shapes=[
            pltpu.VMEM((N, DIM), jnp.float32),
            pltpu.VMEM((N, E_ATTN * HEAD_DIM), jnp.bfloat16),
            pltpu.VMEM((N, HEAD_DIM), jnp.bfloat16),
            pltpu.VMEM((N, HEAD_DIM), jnp.bfloat16),
            pltpu.VMEM((N, E_ATTN), jnp.float32),
            pltpu.VMEM((N, DIM), jnp.bfloat16),
            pltpu.VMEM((N, E_FFD), jnp.float32),
            pltpu.VMEM((TILE, E_ATTN * HEAD_DIM), jnp.bfloat16),
        ],
    )(x2d, ln1_g.reshape(1, DIM), ln1_b.reshape(1, DIM), wg_a,
      w_kv, b_kv.reshape(1, 2 * HEAD_DIM), w_mapf, b_mapf,
      w_outf, b_out, ln2_g.reshape(1, DIM), ln2_b.reshape(1, DIM), wg_m,
      w1, b1.reshape(E_FFD, 1, DIM), w2, b2)

    return out.reshape(x.shape)


# speedup vs baseline: 1.0139x; 1.0139x over previous
import jax
import jax.numpy as jnp
from jax.experimental import pallas as pl
from jax.experimental.pallas import tpu as pltpu

N = 2048
DIM = 768
HEAD_DIM = 64
E_ATTN = 16
E_FFD = 8
FFD_K = 2
N_HEADS = 12
SCALE = HEAD_DIM ** -0.5
TILE = 512
NT = N // TILE


def _topk_gates_dense(logits, k):
    t, e = logits.shape
    eidx = jax.lax.broadcasted_iota(jnp.int32, (t, e), 1)
    rank = jnp.zeros((t, e), jnp.int32)
    for j in range(e):
        lj = logits[:, j:j + 1]
        beats = (lj > logits) | ((lj == logits) & (j < eidx))
        rank += beats.astype(jnp.int32)
    mask = rank < k
    m = jnp.max(logits, axis=-1, keepdims=True)
    ex = jnp.where(mask, jnp.exp(logits - m), 0.0)
    return ex / jnp.sum(ex, axis=-1, keepdims=True)


def _layer_norm(x, g, b):
    mu = jnp.mean(x, axis=-1, keepdims=True)
    var = jnp.mean((x - mu) ** 2, axis=-1, keepdims=True)
    return (x - mu) * jax.lax.rsqrt(var + 1e-5) * g + b


def _kernel(x_ref, g1_ref, b1l_ref, wga_ref, wkv_ref, bkv_ref, wmap_ref,
            bmap_ref, wout_ref, bout_ref, g2_ref, b2l_ref, wgm_ref,
            w1_ref, b1_ref, w2_ref, b2f_ref,
            out_ref,
            x1_s, qall_s, k_s, v_s, g16_s, xn2_s, g8_s, o16_s):
    t = pl.program_id(0)

    @pl.when(t < NT)
    def _pre():
        rows = pl.ds(t * TILE, TILE)
        x = x_ref[...]
        x1_s[rows, :] = x
        xn = _layer_norm(x, g1_ref[...], b1l_ref[...])
        xnb = xn.astype(jnp.bfloat16)
        logits = jnp.dot(xn, wga_ref[...], preferred_element_type=jnp.float32)
        g16_s[rows, :] = _topk_gates_dense(logits, N_HEADS)
        kv = jnp.dot(xnb, wkv_ref[...].astype(jnp.bfloat16),
                     preferred_element_type=jnp.float32) + bkv_ref[...]
        k_s[rows, :] = kv[:, :HEAD_DIM].astype(jnp.bfloat16)
        v_s[rows, :] = kv[:, HEAD_DIM:].astype(jnp.bfloat16)
        qall = jnp.dot(xnb, wmap_ref[...].astype(jnp.bfloat16),
                       preferred_element_type=jnp.float32) + bmap_ref[...]
        qall_s[rows, :] = (qall * (SCALE * 1.4426950408889634)
                           ).astype(jnp.bfloat16)

    @pl.when((t >= NT) & (t < 2 * NT))
    def _attn():
        rows = pl.ds((t - NT) * TILE, TILE)
        qall = qall_s[rows, :]
        k = k_s[...]
        v = v_s[...]
        g16 = g16_s[rows, :]
        for e in range(E_ATTN):
            q = qall[:, e * HEAD_DIM:(e + 1) * HEAD_DIM]
            s = jax.lax.dot_general(q, k, (((1,), (1,)), ((), ())),
                                    preferred_element_type=jnp.float32)
            p = jnp.exp2(s)
            denom = jnp.sum(p, axis=-1, keepdims=True)
            o = jnp.dot(p.astype(jnp.bfloat16), v,
                        preferred_element_type=jnp.float32) / denom
            o16_s[:, e * HEAD_DIM:(e + 1) * HEAD_DIM] = (
                o * g16[:, e:e + 1]).astype(jnp.bfloat16)
        y = jnp.dot(o16_s[...], wout_ref[...].astype(jnp.bfloat16),
                    preferred_element_type=jnp.float32)
        y = y + jnp.dot(g16, bout_ref[...],
                        preferred_element_type=jnp.float32)
        x1 = x1_s[rows, :] + y
        x1_s[rows, :] = x1
        xn2 = _layer_norm(x1, g2_ref[...], b2l_ref[...])
        xn2_s[rows, :] = xn2.astype(jnp.bfloat16)
        logits = jnp.dot(xn2, wgm_ref[...], preferred_element_type=jnp.float32)
        g8_s[rows, :] = _topk_gates_dense(logits, FFD_K)

    @pl.when(t >= 2 * NT)
    def _ffn():
        e = t - 2 * NT
        xn2 = xn2_s[...]
        g8 = g8_s[...]
        h = jnp.dot(xn2, w1_ref[0].astype(jnp.bfloat16),
                    preferred_element_type=jnp.float32)
        h = h + b1_ref[0]
        inner = h * (0.7978845608028654 + 0.035677408136300125 * (h * h))
        h = 0.5 * h * (1.0 + jnp.tanh(inner))
        sel = (jax.lax.broadcasted_iota(jnp.int32, (E_FFD, 1), 0) == e
               ).astype(jnp.float32)
        g = jnp.dot(g8, sel, preferred_element_type=jnp.float32)
        hw = (h * g).astype(jnp.bfloat16)
        acc = jnp.dot(hw, w2_ref[0].astype(jnp.bfloat16),
                      preferred_element_type=jnp.float32)

        @pl.when(e == 0)
        def _init():
            out_ref[...] = x1_s[...] + jnp.dot(
                g8, b2f_ref[...], preferred_element_type=jnp.float32) + acc

        @pl.when(e != 0)
        def _acc():
            out_ref[...] = out_ref[...] + acc


def _full(shape):
    n = len(shape)
    return pl.BlockSpec(shape, lambda *_: (0,) * n)


def kernel(x, task_bh, ln1_g, ln1_b, ln2_g, ln2_b, wg_attn, w_map, b_map,
           w_out, b_out, w_kv, b_kv, wg_mlp, w1, b1, w2, b2):
    x2d = x.reshape(N, DIM)
    wg_a = jax.lax.dynamic_index_in_dim(wg_attn, task_bh, 0, keepdims=False)
    wg_m = jax.lax.dynamic_index_in_dim(wg_mlp, task_bh, 0, keepdims=False)
    w_mapf = jnp.transpose(w_map, (1, 0, 2)).reshape(DIM, E_ATTN * HEAD_DIM)
    b_mapf = b_map.reshape(1, E_ATTN * HEAD_DIM)
    w_outf = w_out.reshape(E_ATTN * HEAD_DIM, DIM)

    def _xmap(t):
        return (jnp.minimum(t, NT - 1), 0)

    def _emap3(t):
        return (jnp.clip(t - 2 * NT, 0, E_FFD - 1), 0, 0)

    out = pl.pallas_call(
        _kernel,
        grid=(2 * NT + E_FFD,),
        in_specs=[
            pl.BlockSpec((TILE, DIM), _xmap),
            _full((1, DIM)), _full((1, DIM)),
            _full((DIM, E_ATTN)),
            _full((DIM, 2 * HEAD_DIM)), _full((1, 2 * HEAD_DIM)),
            _full((DIM, E_ATTN * HEAD_DIM)), _full((1, E_ATTN * HEAD_DIM)),
            _full((E_ATTN * HEAD_DIM, DIM)), _full((E_ATTN, DIM)),
            _full((1, DIM)), _full((1, DIM)),
            _full((DIM, E_FFD)),
            pl.BlockSpec((1, DIM, DIM), _emap3),
            pl.BlockSpec((1, 1, DIM), _emap3),
            pl.BlockSpec((1, DIM, DIM), _emap3),
            _full((E_FFD, DIM)),
        ],
        out_specs=_full((N, DIM)),
        out_shape=jax.ShapeDtypeStruct((N, DIM), jnp.float32),
        scratch_shapes=[
            pltpu.VMEM((N, DIM), jnp.float32),
            pltpu.VMEM((N, E_ATTN * HEAD_DIM), jnp.bfloat16),
            pltpu.VMEM((N, HEAD_DIM), jnp.bfloat16),
            pltpu.VMEM((N, HEAD_DIM), jnp.bfloat16),
            pltpu.VMEM((N, E_ATTN), jnp.float32),
            pltpu.VMEM((N, DIM), jnp.bfloat16),
            pltpu.VMEM((N, E_FFD), jnp.float32),
            pltpu.VMEM((TILE, E_ATTN * HEAD_DIM), jnp.bfloat16),
        ],
    )(x2d, ln1_g.reshape(1, DIM), ln1_b.reshape(1, DIM), wg_a,
      w_kv, b_kv.reshape(1, 2 * HEAD_DIM), w_mapf, b_mapf,
      w_outf, b_out, ln2_g.reshape(1, DIM), ln2_b.reshape(1, DIM), wg_m,
      w1, b1.reshape(E_FFD, 1, DIM), w2, b2)

    return out.reshape(x.shape)
